# baseline (speedup 1.0000x reference)
import jax
import jax.numpy as jnp
from jax import lax
from jax.experimental import pallas as pl
from jax.experimental.pallas import tpu as pltpu

NEG_RATIO = 3
INT_MIN32 = -2147483648


def _ssd_body(lt_ref, li_ref, bi_ref, bt_ref, p4_ref, out_ref):
    R, C, A = li_ref.shape

    tt = lt_ref[...]
    pos = tt > 0
    posf = pos.astype(jnp.float32)
    npos_row = jnp.sum(posf, axis=1, keepdims=True)

    d = bi_ref[...] - bt_ref[...]
    ad = jnp.abs(d)
    sl1 = jnp.where(ad < 1.0, 0.5 * d * d, ad - 0.5)
    m4 = p4_ref[...].astype(jnp.float32)
    bbox_row = jnp.sum(sl1 * m4, axis=1, keepdims=True)

    li = li_ref[...]
    lvl = [li[:, c, :] for c in range(C)]
    bit = 0
    while len(lvl) > 1:
        sel = (tt & (1 << bit)) != 0
        nxt = []
        for j in range(0, len(lvl) - 1, 2):
            nxt.append(jnp.where(sel, lvl[j + 1], lvl[j]))
        if len(lvl) % 2 == 1:
            nxt.append(lvl[-1])
        lvl = nxt
        bit += 1
    ll = -lvl[0]

    masked = jnp.where(pos, 0.0, -ll)
    b = lax.bitcast_convert_type(masked, jnp.int32)
    keys = jnp.where(b >= 0, b, INT_MIN32 - b)

    K = jnp.minimum(
        NEG_RATIO * jnp.sum(pos.astype(jnp.int32), axis=1, keepdims=True),
        A).astype(jnp.int32)

    lo0 = jnp.full((R, 1), INT_MIN32, jnp.int32)
    hi0 = jnp.full((R, 1), 2**31 - 1, jnp.int32)

    def bisect(_, carry):
        lo, hi = carry
        mid = lo + lax.shift_right_logical(hi - lo, 1)
        cnt = jnp.sum((keys <= mid).astype(jnp.int32), axis=1, keepdims=True)
        take = cnt >= K
        return jnp.where(take, lo, mid + 1), jnp.where(take, mid, hi)

    _, thresh = lax.fori_loop(0, 32, bisect, (lo0, hi0))

    below = keys < thresh
    cnt_below = jnp.sum(below.astype(jnp.int32), axis=1, keepdims=True)
    sum_below = jnp.sum(jnp.where(below & ~pos, ll, 0.0), axis=1, keepdims=True)
    tb = jnp.where(thresh >= 0, thresh, INT_MIN32 - thresh)
    tf = lax.bitcast_convert_type(tb, jnp.float32)
    neg_sum = sum_below + (K - cnt_below).astype(jnp.float32) * (-tf)
    neg_sum = jnp.where(K > 0, neg_sum, 0.0)

    label_row = jnp.sum(ll * posf, axis=1, keepdims=True) + neg_sum

    col = lax.broadcasted_iota(jnp.int32, (R, 128), 1)
    out_ref[...] = (jnp.where(col == 0, bbox_row, 0.0)
                    + jnp.where(col == 1, label_row, 0.0)
                    + jnp.where(col == 2, npos_row, 0.0))


def kernel(bbox_input, label_input, bbox_target, label_target):
    B, C, A = label_input.shape
    R = 16
    lt = label_target.astype(jnp.int32)
    pos4 = jnp.broadcast_to((lt > 0)[:, :, None], (B, A, 4))
    pos4 = pos4.reshape(B, 4 * A).astype(jnp.bfloat16)
    bi2 = bbox_input.reshape(B, 4 * A)
    bt2 = bbox_target.reshape(B, 4 * A)

    stats = pl.pallas_call(
        _ssd_body,
        grid=(B // R,),
        in_specs=[
            pl.BlockSpec((R, A), lambda i: (i, 0)),
            pl.BlockSpec((R, C, A), lambda i: (i, 0, 0)),
            pl.BlockSpec((R, 4 * A), lambda i: (i, 0)),
            pl.BlockSpec((R, 4 * A), lambda i: (i, 0)),
            pl.BlockSpec((R, 4 * A), lambda i: (i, 0)),
        ],
        out_specs=pl.BlockSpec((R, 128), lambda i: (i, 0)),
        out_shape=jax.ShapeDtypeStruct((B, 128), jnp.float32),
    )(lt, label_input, bi2, bt2, pos4)

    num_pos = jnp.sum(stats[:, 2])
    return (jnp.sum(stats[:, 0]) + jnp.sum(stats[:, 1])) / num_pos


# speedup vs baseline: 1.0008x; 1.0008x over previous
import jax
import jax.numpy as jnp
from jax import lax
from jax.experimental import pallas as pl
from jax.experimental.pallas import tpu as pltpu

NEG_RATIO = 3
INT_MIN32 = -2147483648


def _bbox_body(bi_ref, bt_ref, p4_ref, out_ref):
    R = bi_ref.shape[0]
    d = bi_ref[...] - bt_ref[...]
    ad = jnp.abs(d)
    sl1 = jnp.where(ad < 1.0, 0.5 * d * d, ad - 0.5)
    m4 = p4_ref[...].astype(jnp.float32)
    bbox_row = jnp.sum(sl1 * m4, axis=1, keepdims=True)
    col = lax.broadcasted_iota(jnp.int32, (R, 128), 1)
    out_ref[...] = jnp.where(col == 0, bbox_row, 0.0)


def _label_body(lt_ref, li_ref, out_ref):
    R, C, A = li_ref.shape

    tt = lt_ref[...]
    pos = tt > 0
    posf = pos.astype(jnp.float32)
    npos_row = jnp.sum(posf, axis=1, keepdims=True)

    li = li_ref[...]
    lvl = [li[:, c, :] for c in range(C)]
    bit = 0
    while len(lvl) > 1:
        sel = (tt & (1 << bit)) != 0
        nxt = []
        for j in range(0, len(lvl) - 1, 2):
            nxt.append(jnp.where(sel, lvl[j + 1], lvl[j]))
        if len(lvl) % 2 == 1:
            nxt.append(lvl[-1])
        lvl = nxt
        bit += 1
    ll = -lvl[0]

    masked = jnp.where(pos, 0.0, -ll)
    b = lax.bitcast_convert_type(masked, jnp.int32)
    keys = jnp.where(b >= 0, b, INT_MIN32 - b)

    K = jnp.minimum(
        NEG_RATIO * jnp.sum(pos.astype(jnp.int32), axis=1, keepdims=True),
        A).astype(jnp.int32)

    lo0 = jnp.full((R, 1), INT_MIN32, jnp.int32)
    hi0 = jnp.full((R, 1), 2**31 - 1, jnp.int32)

    def bisect(_, carry):
        lo, hi = carry
        mid = lo + lax.shift_right_logical(hi - lo, 1)
        cnt = jnp.sum((keys <= mid).astype(jnp.int32), axis=1, keepdims=True)
        take = cnt >= K
        return jnp.where(take, lo, mid + 1), jnp.where(take, mid, hi)

    _, thresh = lax.fori_loop(0, 32, bisect, (lo0, hi0))

    below = keys < thresh
    cnt_below = jnp.sum(below.astype(jnp.int32), axis=1, keepdims=True)
    sum_below = jnp.sum(jnp.where(below & ~pos, ll, 0.0), axis=1, keepdims=True)
    tb = jnp.where(thresh >= 0, thresh, INT_MIN32 - thresh)
    tf = lax.bitcast_convert_type(tb, jnp.float32)
    neg_sum = sum_below + (K - cnt_below).astype(jnp.float32) * (-tf)
    neg_sum = jnp.where(K > 0, neg_sum, 0.0)

    label_row = jnp.sum(ll * posf, axis=1, keepdims=True) + neg_sum

    col = lax.broadcasted_iota(jnp.int32, (R, 128), 1)
    out_ref[...] = (jnp.where(col == 0, label_row, 0.0)
                    + jnp.where(col == 1, npos_row, 0.0))


def kernel(bbox_input, label_input, bbox_target, label_target):
    B, C, A = label_input.shape
    RL = 16
    RB = 32
    lt = label_target.astype(jnp.int32)
    pos4 = jnp.broadcast_to((lt > 0)[:, :, None], (B, A, 4))
    pos4 = pos4.reshape(B, 4 * A).astype(jnp.int8)
    bi2 = bbox_input.reshape(B, 4 * A)
    bt2 = bbox_target.reshape(B, 4 * A)

    bstats = pl.pallas_call(
        _bbox_body,
        grid=(B // RB,),
        in_specs=[
            pl.BlockSpec((RB, 4 * A), lambda i: (i, 0)),
            pl.BlockSpec((RB, 4 * A), lambda i: (i, 0)),
            pl.BlockSpec((RB, 4 * A), lambda i: (i, 0)),
        ],
        out_specs=pl.BlockSpec((RB, 128), lambda i: (i, 0)),
        out_shape=jax.ShapeDtypeStruct((B, 128), jnp.float32),
    )(bi2, bt2, pos4)

    lstats = pl.pallas_call(
        _label_body,
        grid=(B // RL,),
        in_specs=[
            pl.BlockSpec((RL, A), lambda i: (i, 0)),
            pl.BlockSpec((RL, C, A), lambda i: (i, 0, 0)),
        ],
        out_specs=pl.BlockSpec((RL, 128), lambda i: (i, 0)),
        out_shape=jax.ShapeDtypeStruct((B, 128), jnp.float32),
    )(lt, label_input)

    num_pos = jnp.sum(lstats[:, 1])
    return (jnp.sum(bstats[:, 0]) + jnp.sum(lstats[:, 0])) / num_pos
